# Initial kernel scaffold; baseline (speedup 1.0000x reference)
import functools

import jax
import jax.numpy as jnp
from jax import lax
from jax.experimental import pallas as pl
from jax.experimental.pallas import tpu as pltpu
from jax.experimental.pallas import tpu_sc as plsc

H = 128
BN = 2000
EK = 40


def _edge_body(n_nodes, n_edges, a_hbm, b_hbm, dst_hbm, src_hbm, bias_hbm,
               out_hbm, aggr_sh, idx_d, idx_s, a_buf, b_buf, bias_v, sem):
    cid = lax.axis_index("c")
    sid = lax.axis_index("s")
    zero = jnp.zeros((16,), jnp.float32)

    def zrow(i, carry):
        for j in range(8):
            a_buf[i, pl.ds(16 * j, 16)] = zero
        return carry
    lax.fori_loop(0, EK, zrow, 0)

    rows_per_tile = n_nodes // 16
    nz = rows_per_tile // EK
    rz = rows_per_tile - nz * EK
    row0 = sid * rows_per_tile

    def zcopy(i, carry):
        pltpu.sync_copy(a_buf, aggr_sh.at[pl.ds(row0 + i * EK, EK)])
        return carry
    lax.fori_loop(0, nz, zcopy, 0)
    if rz:
        pltpu.sync_copy(a_buf.at[pl.ds(0, rz)],
                        aggr_sh.at[pl.ds(row0 + nz * EK, rz)])

    pltpu.sync_copy(bias_hbm, bias_v)
    bvecs = [bias_v[pl.ds(16 * j, 16)] for j in range(8)]

    plsc.subcore_barrier()

    e_per_w = n_edges // 32
    base = cid * (n_edges // 2) + sid * e_per_w
    nchunks = e_per_w // EK

    def chunk(c, carry):
        off = base + c * EK
        pltpu.sync_copy(dst_hbm.at[pl.ds(off, EK)], idx_d)
        pltpu.sync_copy(src_hbm.at[pl.ds(off, EK)], idx_s)
        ca = pltpu.async_copy(a_hbm.at[idx_d], a_buf, sem)
        cb = pltpu.async_copy(b_hbm.at[idx_s], b_buf, sem)
        ca.wait()
        cb.wait()

        def edge(i, carry2):
            for j in range(8):
                sl = pl.ds(16 * j, 16)
                a_buf[i, sl] = jnp.maximum(
                    a_buf[i, sl] + b_buf[i, sl] + bvecs[j], zero)
            return carry2
        lax.fori_loop(0, EK, edge, 0)

        pltpu.sync_copy(a_buf, aggr_sh.at[idx_d], add=True)
        return carry
    lax.fori_loop(0, nchunks, chunk, 0)

    plsc.subcore_barrier()

    pltpu.sync_copy(aggr_sh.at[pl.ds(row0, rows_per_tile)],
                    out_hbm.at[cid, pl.ds(row0, rows_per_tile)])


@functools.lru_cache(maxsize=None)
def _make_edge_kernel(n_nodes, n_edges):
    mesh = plsc.VectorSubcoreMesh(core_axis_name="c", subcore_axis_name="s")
    return pl.kernel(
        functools.partial(_edge_body, n_nodes, n_edges),
        out_type=jax.ShapeDtypeStruct((2, n_nodes, H), jnp.float32),
        mesh=mesh,
        scratch_types=[
            pltpu.VMEM_SHARED((n_nodes, H), jnp.float32),
            pltpu.VMEM((EK,), jnp.int32),
            pltpu.VMEM((EK,), jnp.int32),
            pltpu.VMEM((EK, H), jnp.float32),
            pltpu.VMEM((EK, H), jnp.float32),
            pltpu.VMEM((H,), jnp.float32),
            pltpu.SemaphoreType.DMA,
        ],
    )


def _proj_ab_block(x_ref, w0_ref, b0_ref, wab_ref, oa_ref, ob_ref):
    h = jnp.maximum(
        jnp.dot(x_ref[...], w0_ref[...], preferred_element_type=jnp.float32)
        + b0_ref[...], 0.0)
    ab = jnp.dot(h, wab_ref[...], preferred_element_type=jnp.float32)
    oa_ref[...] = ab[:, :H]
    ob_ref[...] = ab[:, H:]


def _upd_ab_block(p_ref, wl_ref, bl_ref, wab_ref, oa_ref, ob_ref):
    agg = p_ref[0] + p_ref[1]
    h = jnp.maximum(
        jnp.dot(agg, wl_ref[...], preferred_element_type=jnp.float32)
        + bl_ref[...], 0.0)
    ab = jnp.dot(h, wab_ref[...], preferred_element_type=jnp.float32)
    oa_ref[...] = ab[:, :H]
    ob_ref[...] = ab[:, H:]


def _upd_mean_block(p_ref, wl_ref, bl_ref, o_ref, *, n_nodes):
    agg = p_ref[0] + p_ref[1]
    h = jnp.maximum(
        jnp.dot(agg, wl_ref[...], preferred_element_type=jnp.float32)
        + bl_ref[...], 0.0)
    i = pl.program_id(0)

    @pl.when(i == 0)
    def _():
        o_ref[...] = jnp.zeros_like(o_ref)
    o_ref[...] += jnp.sum(h, axis=0, keepdims=True) * (1.0 / n_nodes)


def _head_block(t_ref, g_ref, wmu_ref, bmu_ref, wlv_ref, blv_ref,
                mu_ref, lv_ref, f_ref):
    f = jnp.concatenate([t_ref[...], g_ref[...]], axis=1)
    f_ref[...] = f
    mu_ref[...] = jnp.dot(f, wmu_ref[...],
                          preferred_element_type=jnp.float32) + bmu_ref[...]
    lv_ref[...] = jnp.dot(f, wlv_ref[...],
                          preferred_element_type=jnp.float32) + blv_ref[...]


def _proj_ab(x, w0t, b0, wab, n):
    grid = (n // BN,)
    return pl.pallas_call(
        _proj_ab_block,
        grid=grid,
        in_specs=[
            pl.BlockSpec((BN, H), lambda i: (i, 0)),
            pl.BlockSpec((H, H), lambda i: (0, 0)),
            pl.BlockSpec((1, H), lambda i: (0, 0)),
            pl.BlockSpec((H, 2 * H), lambda i: (0, 0)),
        ],
        out_specs=[
            pl.BlockSpec((BN, H), lambda i: (i, 0)),
            pl.BlockSpec((BN, H), lambda i: (i, 0)),
        ],
        out_shape=[
            jax.ShapeDtypeStruct((n, H), jnp.float32),
            jax.ShapeDtypeStruct((n, H), jnp.float32),
        ],
    )(x, w0t, b0.reshape(1, H), wab)


def _upd_ab(p, wlt, bl, wab, n):
    grid = (n // BN,)
    return pl.pallas_call(
        _upd_ab_block,
        grid=grid,
        in_specs=[
            pl.BlockSpec((2, BN, H), lambda i: (0, i, 0)),
            pl.BlockSpec((H, H), lambda i: (0, 0)),
            pl.BlockSpec((1, H), lambda i: (0, 0)),
            pl.BlockSpec((H, 2 * H), lambda i: (0, 0)),
        ],
        out_specs=[
            pl.BlockSpec((BN, H), lambda i: (i, 0)),
            pl.BlockSpec((BN, H), lambda i: (i, 0)),
        ],
        out_shape=[
            jax.ShapeDtypeStruct((n, H), jnp.float32),
            jax.ShapeDtypeStruct((n, H), jnp.float32),
        ],
    )(p, wlt, bl.reshape(1, H), wab)


def _upd_mean(p, wlt, bl, n):
    grid = (n // BN,)
    return pl.pallas_call(
        functools.partial(_upd_mean_block, n_nodes=n),
        grid=grid,
        in_specs=[
            pl.BlockSpec((2, BN, H), lambda i: (0, i, 0)),
            pl.BlockSpec((H, H), lambda i: (0, 0)),
            pl.BlockSpec((1, H), lambda i: (0, 0)),
        ],
        out_specs=pl.BlockSpec((1, H), lambda i: (0, 0)),
        out_shape=jax.ShapeDtypeStruct((1, H), jnp.float32),
    )(p, wlt, bl.reshape(1, H))


def _heads(tvec, gvec, muW, mub, lvW, lvb):
    z = muW.shape[0]
    return pl.pallas_call(
        _head_block,
        out_shape=[
            jax.ShapeDtypeStruct((1, z), jnp.float32),
            jax.ShapeDtypeStruct((1, z), jnp.float32),
            jax.ShapeDtypeStruct((1, 2 * H), jnp.float32),
        ],
    )(tvec, gvec, muW.T, mub.reshape(1, z), lvW.T, lvb.reshape(1, z))


def _encode_one(x, ei, inW, inb, layers, n_nodes, n_edges):
    dst = ei[1]
    src = ei[0]
    edge_k = _make_edge_kernel(n_nodes, n_edges)

    (m0W, m0b, l0W, l0b), (m1W, m1b, l1W, l1b), (m2W, m2b, l2W, l2b) = layers

    a, b = _proj_ab(x, inW.T, inb, m0W.T, n_nodes)
    p = edge_k(a, b, dst, src, m0b)
    a, b = _upd_ab(p, l0W.T, l0b, m1W.T, n_nodes)
    p = edge_k(a, b, dst, src, m1b)
    a, b = _upd_ab(p, l1W.T, l1b, m2W.T, n_nodes)
    p = edge_k(a, b, dst, src, m2b)
    return _upd_mean(p, l2W.T, l2b, n_nodes)


def kernel(tree_x, graph_x, tree_edge_index, graph_edge_index,
           t_inW, t_inb, t_m0W, t_m0b, t_l0W, t_l0b, t_m1W, t_m1b,
           t_l1W, t_l1b, t_m2W, t_m2b, t_l2W, t_l2b,
           g_inW, g_inb, g_m0W, g_m0b, g_l0W, g_l0b, g_m1W, g_m1b,
           g_l1W, g_l1b, g_m2W, g_m2b, g_l2W, g_l2b,
           muW, mub, lvW, lvb):
    tlayers = [(t_m0W, t_m0b, t_l0W, t_l0b), (t_m1W, t_m1b, t_l1W, t_l1b),
               (t_m2W, t_m2b, t_l2W, t_l2b)]
    glayers = [(g_m0W, g_m0b, g_l0W, g_l0b), (g_m1W, g_m1b, g_l1W, g_l1b),
               (g_m2W, g_m2b, g_l2W, g_l2b)]
    tvec = _encode_one(tree_x, tree_edge_index, t_inW, t_inb, tlayers,
                       tree_x.shape[0], tree_edge_index.shape[1])
    gvec = _encode_one(graph_x, graph_edge_index, g_inW, g_inb, glayers,
                       graph_x.shape[0], graph_edge_index.shape[1])
    mu, logvar, fused = _heads(tvec, gvec, muW, mub, lvW, lvb)
    return (mu, logvar, fused)


# speedup vs baseline: 2.5010x; 2.5010x over previous
import functools

import jax
import jax.numpy as jnp
from jax import lax
from jax.experimental import pallas as pl
from jax.experimental.pallas import tpu as pltpu
from jax.experimental.pallas import tpu_sc as plsc

H = 128
BN = 2000
EK = 40


def _edge_body(n_nodes, n_edges, a_hbm, b_hbm, dst_hbm, src_hbm, bias_hbm,
               out_hbm, aggr_sh, idx_d, idx_s, a_buf, b_buf, bias_v, sem):
    cid = lax.axis_index("c")
    sid = lax.axis_index("s")
    zero = jnp.zeros((16,), jnp.float32)

    def zrow(i, carry):
        for j in range(8):
            a_buf[i, pl.ds(16 * j, 16)] = zero
        return carry
    lax.fori_loop(0, EK, zrow, 0)

    rpt = ((-(-n_nodes // 16) + EK - 1) // EK) * EK
    row0 = sid * rpt
    nrows = jnp.clip(n_nodes - row0, 0, rpt)
    n_blk = nrows // EK

    def zcopy(i, carry):
        pltpu.sync_copy(a_buf, aggr_sh.at[pl.ds(row0 + i * EK, EK)])
        return carry
    lax.fori_loop(0, n_blk, zcopy, 0)

    pltpu.sync_copy(bias_hbm, bias_v)
    bvecs = [bias_v[pl.ds(16 * j, 16)] for j in range(8)]

    plsc.subcore_barrier()

    e_per_w = n_edges // 32
    base = cid * (n_edges // 2) + sid * e_per_w
    nchunks = e_per_w // EK

    def chunk(c, carry):
        off = base + c * EK
        pltpu.sync_copy(dst_hbm.at[pl.ds(off, EK)], idx_d)
        pltpu.sync_copy(src_hbm.at[pl.ds(off, EK)], idx_s)
        ca = pltpu.async_copy(a_hbm.at[idx_d], a_buf, sem)
        cb = pltpu.async_copy(b_hbm.at[idx_s], b_buf, sem)
        ca.wait()
        cb.wait()

        def edge(i, carry2):
            for j in range(8):
                sl = pl.ds(16 * j, 16)
                a_buf[i, sl] = jnp.maximum(
                    a_buf[i, sl] + b_buf[i, sl] + bvecs[j], zero)
            return carry2
        lax.fori_loop(0, EK, edge, 0)

        pltpu.sync_copy(a_buf, aggr_sh.at[idx_d], add=True)
        return carry
    lax.fori_loop(0, nchunks, chunk, 0)

    plsc.subcore_barrier()

    def wcopy(i, carry):
        r = row0 + i * EK
        pltpu.sync_copy(aggr_sh.at[pl.ds(r, EK)],
                        out_hbm.at[cid, pl.ds(r, EK)])
        return carry
    lax.fori_loop(0, n_blk, wcopy, 0)


@functools.lru_cache(maxsize=None)
def _make_edge_kernel(n_nodes, n_edges):
    mesh = plsc.VectorSubcoreMesh(core_axis_name="c", subcore_axis_name="s")
    return pl.kernel(
        functools.partial(_edge_body, n_nodes, n_edges),
        out_type=jax.ShapeDtypeStruct((2, n_nodes, H), jnp.float32),
        mesh=mesh,
        scratch_types=[
            pltpu.VMEM_SHARED((n_nodes, H), jnp.float32),
            pltpu.VMEM((EK,), jnp.int32),
            pltpu.VMEM((EK,), jnp.int32),
            pltpu.VMEM((EK, H), jnp.float32),
            pltpu.VMEM((EK, H), jnp.float32),
            pltpu.VMEM((H,), jnp.float32),
            pltpu.SemaphoreType.DMA,
        ],
    )


def _proj_ab_block(x_ref, w0_ref, b0_ref, wab_ref, oa_ref, ob_ref):
    h = jnp.maximum(
        jnp.dot(x_ref[...], w0_ref[...], preferred_element_type=jnp.float32)
        + b0_ref[...], 0.0)
    ab = jnp.dot(h, wab_ref[...], preferred_element_type=jnp.float32)
    oa_ref[...] = ab[:, :H]
    ob_ref[...] = ab[:, H:]


def _upd_ab_block(p_ref, wl_ref, bl_ref, wab_ref, oa_ref, ob_ref):
    agg = p_ref[0] + p_ref[1]
    h = jnp.maximum(
        jnp.dot(agg, wl_ref[...], preferred_element_type=jnp.float32)
        + bl_ref[...], 0.0)
    ab = jnp.dot(h, wab_ref[...], preferred_element_type=jnp.float32)
    oa_ref[...] = ab[:, :H]
    ob_ref[...] = ab[:, H:]


def _upd_mean_block(p_ref, wl_ref, bl_ref, o_ref, *, n_nodes):
    agg = p_ref[0] + p_ref[1]
    h = jnp.maximum(
        jnp.dot(agg, wl_ref[...], preferred_element_type=jnp.float32)
        + bl_ref[...], 0.0)
    i = pl.program_id(0)

    @pl.when(i == 0)
    def _():
        o_ref[...] = jnp.zeros_like(o_ref)
    o_ref[...] += jnp.sum(h, axis=0, keepdims=True) * (1.0 / n_nodes)


def _head_block(t_ref, g_ref, wmu_ref, bmu_ref, wlv_ref, blv_ref,
                mu_ref, lv_ref, f_ref):
    f = jnp.concatenate([t_ref[...], g_ref[...]], axis=1)
    f_ref[...] = f
    mu_ref[...] = jnp.dot(f, wmu_ref[...],
                          preferred_element_type=jnp.float32) + bmu_ref[...]
    lv_ref[...] = jnp.dot(f, wlv_ref[...],
                          preferred_element_type=jnp.float32) + blv_ref[...]


def _proj_ab(x, w0t, b0, wab, n):
    grid = (n // BN,)
    return pl.pallas_call(
        _proj_ab_block,
        grid=grid,
        in_specs=[
            pl.BlockSpec((BN, H), lambda i: (i, 0)),
            pl.BlockSpec((H, H), lambda i: (0, 0)),
            pl.BlockSpec((1, H), lambda i: (0, 0)),
            pl.BlockSpec((H, 2 * H), lambda i: (0, 0)),
        ],
        out_specs=[
            pl.BlockSpec((BN, H), lambda i: (i, 0)),
            pl.BlockSpec((BN, H), lambda i: (i, 0)),
        ],
        out_shape=[
            jax.ShapeDtypeStruct((n, H), jnp.float32),
            jax.ShapeDtypeStruct((n, H), jnp.float32),
        ],
    )(x, w0t, b0.reshape(1, H), wab)


def _upd_ab(p, wlt, bl, wab, n):
    grid = (n // BN,)
    return pl.pallas_call(
        _upd_ab_block,
        grid=grid,
        in_specs=[
            pl.BlockSpec((2, BN, H), lambda i: (0, i, 0)),
            pl.BlockSpec((H, H), lambda i: (0, 0)),
            pl.BlockSpec((1, H), lambda i: (0, 0)),
            pl.BlockSpec((H, 2 * H), lambda i: (0, 0)),
        ],
        out_specs=[
            pl.BlockSpec((BN, H), lambda i: (i, 0)),
            pl.BlockSpec((BN, H), lambda i: (i, 0)),
        ],
        out_shape=[
            jax.ShapeDtypeStruct((n, H), jnp.float32),
            jax.ShapeDtypeStruct((n, H), jnp.float32),
        ],
    )(p, wlt, bl.reshape(1, H), wab)


def _upd_mean(p, wlt, bl, n):
    grid = (n // BN,)
    return pl.pallas_call(
        functools.partial(_upd_mean_block, n_nodes=n),
        grid=grid,
        in_specs=[
            pl.BlockSpec((2, BN, H), lambda i: (0, i, 0)),
            pl.BlockSpec((H, H), lambda i: (0, 0)),
            pl.BlockSpec((1, H), lambda i: (0, 0)),
        ],
        out_specs=pl.BlockSpec((1, H), lambda i: (0, 0)),
        out_shape=jax.ShapeDtypeStruct((1, H), jnp.float32),
    )(p, wlt, bl.reshape(1, H))


def _heads(tvec, gvec, muW, mub, lvW, lvb):
    z = muW.shape[0]
    return pl.pallas_call(
        _head_block,
        out_shape=[
            jax.ShapeDtypeStruct((1, z), jnp.float32),
            jax.ShapeDtypeStruct((1, z), jnp.float32),
            jax.ShapeDtypeStruct((1, 2 * H), jnp.float32),
        ],
    )(tvec, gvec, muW.T, mub.reshape(1, z), lvW.T, lvb.reshape(1, z))


def _ab_weight(mW):
    return jnp.concatenate([mW[:, :H].T, mW[:, H:].T], axis=1)


def _encode_one(x, ei, inW, inb, layers, n_nodes, n_edges):
    dst = ei[1]
    src = ei[0]
    edge_k = _make_edge_kernel(n_nodes, n_edges)

    (m0W, m0b, l0W, l0b), (m1W, m1b, l1W, l1b), (m2W, m2b, l2W, l2b) = layers

    a, b = _proj_ab(x, inW.T, inb, _ab_weight(m0W), n_nodes)
    p = edge_k(a, b, dst, src, m0b)
    a, b = _upd_ab(p, l0W.T, l0b, _ab_weight(m1W), n_nodes)
    p = edge_k(a, b, dst, src, m1b)
    a, b = _upd_ab(p, l1W.T, l1b, _ab_weight(m2W), n_nodes)
    p = edge_k(a, b, dst, src, m2b)
    return _upd_mean(p, l2W.T, l2b, n_nodes)


def kernel(tree_x, graph_x, tree_edge_index, graph_edge_index,
           t_inW, t_inb, t_m0W, t_m0b, t_l0W, t_l0b, t_m1W, t_m1b,
           t_l1W, t_l1b, t_m2W, t_m2b, t_l2W, t_l2b,
           g_inW, g_inb, g_m0W, g_m0b, g_l0W, g_l0b, g_m1W, g_m1b,
           g_l1W, g_l1b, g_m2W, g_m2b, g_l2W, g_l2b,
           muW, mub, lvW, lvb):
    tlayers = [(t_m0W, t_m0b, t_l0W, t_l0b), (t_m1W, t_m1b, t_l1W, t_l1b),
               (t_m2W, t_m2b, t_l2W, t_l2b)]
    glayers = [(g_m0W, g_m0b, g_l0W, g_l0b), (g_m1W, g_m1b, g_l1W, g_l1b),
               (g_m2W, g_m2b, g_l2W, g_l2b)]
    tvec = _encode_one(tree_x, tree_edge_index, t_inW, t_inb, tlayers,
                       tree_x.shape[0], tree_edge_index.shape[1])
    gvec = _encode_one(graph_x, graph_edge_index, g_inW, g_inb, glayers,
                       graph_x.shape[0], graph_edge_index.shape[1])
    mu, logvar, fused = _heads(tvec, gvec, muW, mub, lvW, lvb)
    return (mu, logvar, fused)


# speedup vs baseline: 4.4709x; 1.7876x over previous
import functools

import jax
import jax.numpy as jnp
from jax import lax
from jax.experimental import pallas as pl
from jax.experimental.pallas import tpu as pltpu
from jax.experimental.pallas import tpu_sc as plsc

H = 128
BN = 2000


def _edge_body(n_nodes, e_per_w, a_hbm, b_hbm, dst_hbm, src_hbm, bias_hbm,
               out_hbm, aggr_sh, idxd, idxd_g, idxs,
               a0, a1, a2, a3, b0, b1, b2, b3, bias_v,
               sg0, sg1, sg2, sg3, ss0, ss1, ss2, ss3):
    cid = lax.axis_index("c")
    sid = lax.axis_index("s")
    zero = jnp.zeros((16,), jnp.float32)
    A = [a0, a1, a2, a3]
    B = [b0, b1, b2, b3]
    SG = [sg0, sg1, sg2, sg3]
    SS = [ss0, ss1, ss2, ss3]
    nfull = e_per_w // 16
    rem = e_per_w - nfull * 16
    nchunks = nfull + (1 if rem else 0)

    def zrow(i, carry):
        for j in range(8):
            a0[i, pl.ds(16 * j, 16)] = zero
        return carry
    lax.fori_loop(0, 16, zrow, 0)

    rpt = ((-(-n_nodes // 16) + 15) // 16) * 16
    row0 = sid * rpt
    nrows = jnp.clip(n_nodes - row0, 0, rpt)
    n_blk = nrows // 16

    def zcopy(i, carry):
        pltpu.sync_copy(a0, aggr_sh.at[pl.ds(row0 + i * 16, 16)])
        return carry
    lax.fori_loop(0, n_blk, zcopy, 0)

    pltpu.sync_copy(bias_hbm, bias_v)
    bvecs = [bias_v[pl.ds(16 * j, 16)] for j in range(8)]

    base = cid * (e_per_w * 16) + sid * e_per_w
    pltpu.sync_copy(dst_hbm.at[pl.ds(base, e_per_w)],
                    idxd.at[pl.ds(0, e_per_w)])
    pltpu.sync_copy(dst_hbm.at[pl.ds(base, e_per_w)],
                    idxd_g.at[pl.ds(0, e_per_w)])
    pltpu.sync_copy(src_hbm.at[pl.ds(base, e_per_w)],
                    idxs.at[pl.ds(0, e_per_w)])
    if rem:
        lane = lax.iota(jnp.int32, 16)
        off = nfull * 16
        vd = idxd[pl.ds(off, 16)]
        idxd[pl.ds(off, 16)] = jnp.where(lane < rem, vd, n_nodes)
        vg = idxd_g[pl.ds(off, 16)]
        idxd_g[pl.ds(off, 16)] = jnp.where(lane < rem, vg, 0)
        vs = idxs[pl.ds(off, 16)]
        idxs[pl.ds(off, 16)] = jnp.where(lane < rem, vs, 0)

    def stage(c, k):
        pltpu.async_copy(a_hbm.at[idxd_g[pl.ds(c * 16, 16)]], A[k], SG[k])
        pltpu.async_copy(b_hbm.at[idxs[pl.ds(c * 16, 16)]], B[k], SG[k])

    def wait_gather(c, k):
        pltpu.make_async_copy(a_hbm.at[idxd_g[pl.ds(c * 16, 16)]],
                              A[k], SG[k]).wait()
        pltpu.make_async_copy(b_hbm.at[idxs[pl.ds(c * 16, 16)]],
                              B[k], SG[k]).wait()

    def compute(k):
        ab, bb = A[k], B[k]

        def edge(i, carry):
            for j in range(8):
                sl = pl.ds(16 * j, 16)
                ab[i, sl] = jnp.maximum(ab[i, sl] + bb[i, sl] + bvecs[j],
                                        zero)
            return carry
        lax.fori_loop(0, 16, edge, 0)

    def scatter(c, k):
        pltpu.async_copy(A[k], aggr_sh.at[idxd[pl.ds(c * 16, 16)]],
                         SS[k], add=True)

    def wait_scatter(c, k):
        pltpu.make_async_copy(A[k], aggr_sh.at[idxd[pl.ds(c * 16, 16)]],
                              SS[k]).wait()

    def do_step(c, kc, do_stage, do_wait):
        kn = (kc + 2) % 4
        wait_gather(c, kc)
        compute(kc)
        scatter(c, kc)
        if do_wait:
            wait_scatter(c - 2, kn)
        if do_stage:
            stage(c + 2, kn)

    stage(0, 0)
    stage(1, 1)
    plsc.subcore_barrier()

    do_step(0, 0, True, False)
    do_step(1, 1, True, False)
    ngroups = (nchunks - 4) // 4

    def group(g, carry):
        c0 = 2 + 4 * g
        for r in range(4):
            do_step(c0 + r, (2 + r) % 4, True, True)
        return carry
    lax.fori_loop(0, ngroups, group, 0)
    for c in range(2 + 4 * ngroups, nchunks - 2):
        do_step(c, c % 4, True, True)
    for c in range(max(nchunks - 2, 2 + 4 * ngroups), nchunks):
        do_step(c, c % 4, False, False)
    for c in range(nchunks - 4, nchunks):
        wait_scatter(c, c % 4)

    plsc.subcore_barrier()

    def wcopy(i, carry):
        r = row0 + i * 16
        pltpu.sync_copy(aggr_sh.at[pl.ds(r, 16)],
                        out_hbm.at[cid, pl.ds(r, 16)])
        return carry
    lax.fori_loop(0, n_blk, wcopy, 0)


@functools.lru_cache(maxsize=None)
def _make_edge_kernel(n_nodes, n_edges):
    e_per_w = n_edges // 32
    assert e_per_w * 32 == n_edges and e_per_w % 8 == 0
    idx_len = ((e_per_w + 15) // 16) * 16
    mesh = plsc.VectorSubcoreMesh(core_axis_name="c", subcore_axis_name="s")
    return pl.kernel(
        functools.partial(_edge_body, n_nodes, e_per_w),
        out_type=jax.ShapeDtypeStruct((2, n_nodes, H), jnp.float32),
        mesh=mesh,
        scratch_types=(
            [pltpu.VMEM_SHARED((n_nodes + 16, H), jnp.float32)]
            + [pltpu.VMEM((idx_len,), jnp.int32)] * 3
            + [pltpu.VMEM((16, H), jnp.float32)] * 8
            + [pltpu.VMEM((H,), jnp.float32)]
            + [pltpu.SemaphoreType.DMA] * 8
        ),
    )


def _proj_ab_block(x_ref, w0_ref, b0_ref, wab_ref, oa_ref, ob_ref):
    h = jnp.maximum(
        jnp.dot(x_ref[...], w0_ref[...], preferred_element_type=jnp.float32)
        + b0_ref[...], 0.0)
    ab = jnp.dot(h, wab_ref[...], preferred_element_type=jnp.float32)
    oa_ref[...] = ab[:, :H]
    ob_ref[...] = ab[:, H:]


def _upd_ab_block(p_ref, wl_ref, bl_ref, wab_ref, oa_ref, ob_ref):
    agg = p_ref[0] + p_ref[1]
    h = jnp.maximum(
        jnp.dot(agg, wl_ref[...], preferred_element_type=jnp.float32)
        + bl_ref[...], 0.0)
    ab = jnp.dot(h, wab_ref[...], preferred_element_type=jnp.float32)
    oa_ref[...] = ab[:, :H]
    ob_ref[...] = ab[:, H:]


def _upd_mean_block(p_ref, wl_ref, bl_ref, o_ref, *, n_nodes):
    agg = p_ref[0] + p_ref[1]
    h = jnp.maximum(
        jnp.dot(agg, wl_ref[...], preferred_element_type=jnp.float32)
        + bl_ref[...], 0.0)
    i = pl.program_id(0)

    @pl.when(i == 0)
    def _():
        o_ref[...] = jnp.zeros_like(o_ref)
    o_ref[...] += jnp.sum(h, axis=0, keepdims=True) * (1.0 / n_nodes)


def _head_block(t_ref, g_ref, wmu_ref, bmu_ref, wlv_ref, blv_ref,
                mu_ref, lv_ref, f_ref):
    f = jnp.concatenate([t_ref[...], g_ref[...]], axis=1)
    f_ref[...] = f
    mu_ref[...] = jnp.dot(f, wmu_ref[...],
                          preferred_element_type=jnp.float32) + bmu_ref[...]
    lv_ref[...] = jnp.dot(f, wlv_ref[...],
                          preferred_element_type=jnp.float32) + blv_ref[...]


def _proj_ab(x, w0t, b0, wab, n):
    grid = (n // BN,)
    return pl.pallas_call(
        _proj_ab_block,
        grid=grid,
        in_specs=[
            pl.BlockSpec((BN, H), lambda i: (i, 0)),
            pl.BlockSpec((H, H), lambda i: (0, 0)),
            pl.BlockSpec((1, H), lambda i: (0, 0)),
            pl.BlockSpec((H, 2 * H), lambda i: (0, 0)),
        ],
        out_specs=[
            pl.BlockSpec((BN, H), lambda i: (i, 0)),
            pl.BlockSpec((BN, H), lambda i: (i, 0)),
        ],
        out_shape=[
            jax.ShapeDtypeStruct((n, H), jnp.float32),
            jax.ShapeDtypeStruct((n, H), jnp.float32),
        ],
    )(x, w0t, b0.reshape(1, H), wab)


def _upd_ab(p, wlt, bl, wab, n):
    grid = (n // BN,)
    return pl.pallas_call(
        _upd_ab_block,
        grid=grid,
        in_specs=[
            pl.BlockSpec((2, BN, H), lambda i: (0, i, 0)),
            pl.BlockSpec((H, H), lambda i: (0, 0)),
            pl.BlockSpec((1, H), lambda i: (0, 0)),
            pl.BlockSpec((H, 2 * H), lambda i: (0, 0)),
        ],
        out_specs=[
            pl.BlockSpec((BN, H), lambda i: (i, 0)),
            pl.BlockSpec((BN, H), lambda i: (i, 0)),
        ],
        out_shape=[
            jax.ShapeDtypeStruct((n, H), jnp.float32),
            jax.ShapeDtypeStruct((n, H), jnp.float32),
        ],
    )(p, wlt, bl.reshape(1, H), wab)


def _upd_mean(p, wlt, bl, n):
    grid = (n // BN,)
    return pl.pallas_call(
        functools.partial(_upd_mean_block, n_nodes=n),
        grid=grid,
        in_specs=[
            pl.BlockSpec((2, BN, H), lambda i: (0, i, 0)),
            pl.BlockSpec((H, H), lambda i: (0, 0)),
            pl.BlockSpec((1, H), lambda i: (0, 0)),
        ],
        out_specs=pl.BlockSpec((1, H), lambda i: (0, 0)),
        out_shape=jax.ShapeDtypeStruct((1, H), jnp.float32),
    )(p, wlt, bl.reshape(1, H))


def _heads(tvec, gvec, muW, mub, lvW, lvb):
    z = muW.shape[0]
    return pl.pallas_call(
        _head_block,
        out_shape=[
            jax.ShapeDtypeStruct((1, z), jnp.float32),
            jax.ShapeDtypeStruct((1, z), jnp.float32),
            jax.ShapeDtypeStruct((1, 2 * H), jnp.float32),
        ],
    )(tvec, gvec, muW.T, mub.reshape(1, z), lvW.T, lvb.reshape(1, z))


def _ab_weight(mW):
    return jnp.concatenate([mW[:, :H].T, mW[:, H:].T], axis=1)


def _encode_one(x, ei, inW, inb, layers, n_nodes, n_edges):
    dst = ei[1]
    src = ei[0]
    edge_k = _make_edge_kernel(n_nodes, n_edges)

    (m0W, m0b, l0W, l0b), (m1W, m1b, l1W, l1b), (m2W, m2b, l2W, l2b) = layers

    a, b = _proj_ab(x, inW.T, inb, _ab_weight(m0W), n_nodes)
    p = edge_k(a, b, dst, src, m0b)
    a, b = _upd_ab(p, l0W.T, l0b, _ab_weight(m1W), n_nodes)
    p = edge_k(a, b, dst, src, m1b)
    a, b = _upd_ab(p, l1W.T, l1b, _ab_weight(m2W), n_nodes)
    p = edge_k(a, b, dst, src, m2b)
    return _upd_mean(p, l2W.T, l2b, n_nodes)


def kernel(tree_x, graph_x, tree_edge_index, graph_edge_index,
           t_inW, t_inb, t_m0W, t_m0b, t_l0W, t_l0b, t_m1W, t_m1b,
           t_l1W, t_l1b, t_m2W, t_m2b, t_l2W, t_l2b,
           g_inW, g_inb, g_m0W, g_m0b, g_l0W, g_l0b, g_m1W, g_m1b,
           g_l1W, g_l1b, g_m2W, g_m2b, g_l2W, g_l2b,
           muW, mub, lvW, lvb):
    tlayers = [(t_m0W, t_m0b, t_l0W, t_l0b), (t_m1W, t_m1b, t_l1W, t_l1b),
               (t_m2W, t_m2b, t_l2W, t_l2b)]
    glayers = [(g_m0W, g_m0b, g_l0W, g_l0b), (g_m1W, g_m1b, g_l1W, g_l1b),
               (g_m2W, g_m2b, g_l2W, g_l2b)]
    tvec = _encode_one(tree_x, tree_edge_index, t_inW, t_inb, tlayers,
                       tree_x.shape[0], tree_edge_index.shape[1])
    gvec = _encode_one(graph_x, graph_edge_index, g_inW, g_inb, glayers,
                       graph_x.shape[0], graph_edge_index.shape[1])
    mu, logvar, fused = _heads(tvec, gvec, muW, mub, lvW, lvb)
    return (mu, logvar, fused)


# speedup vs baseline: 4.4841x; 1.0030x over previous
import functools

import jax
import jax.numpy as jnp
from jax import lax
from jax.experimental import pallas as pl
from jax.experimental.pallas import tpu as pltpu
from jax.experimental.pallas import tpu_sc as plsc

H = 128
BN = 2000


def _edge_body(n_nodes, e_per_w, a_hbm, b_hbm, dst_hbm, src_hbm, bias_hbm,
               out_hbm, aggr_sh, idxd, idxs,
               a0, a1, a2, a3, b0, b1, b2, b3, o0, o1, o2, o3, bias_v,
               sg0, sg1, sg2, sg3, ss0, ss1, ss2, ss3):
    cid = lax.axis_index("c")
    sid = lax.axis_index("s")
    zero = jnp.zeros((16,), jnp.float32)
    A = [a0, a1, a2, a3]
    B = [b0, b1, b2, b3]
    O = [o0, o1, o2, o3]
    SG = [sg0, sg1, sg2, sg3]
    SS = [ss0, ss1, ss2, ss3]
    nfull = e_per_w // 16
    rem = e_per_w - nfull * 16
    nchunks = nfull + (1 if rem else 0)

    def zrow(i, carry):
        for j in range(8):
            o0[i, pl.ds(16 * j, 16)] = zero
        return carry
    lax.fori_loop(0, 16, zrow, 0)

    rpt = ((-(-n_nodes // 16) + 15) // 16) * 16
    row0 = sid * rpt
    nrows = jnp.clip(n_nodes - row0, 0, rpt)
    n_blk = nrows // 16

    def zcopy(i, carry):
        pltpu.sync_copy(o0, aggr_sh.at[pl.ds(row0 + i * 16, 16)])
        return carry
    lax.fori_loop(0, n_blk, zcopy, 0)

    base = cid * (e_per_w * 16) + sid * e_per_w
    pltpu.sync_copy(dst_hbm.at[pl.ds(base, e_per_w)],
                    idxd.at[pl.ds(0, e_per_w)])
    pltpu.sync_copy(src_hbm.at[pl.ds(base, e_per_w)],
                    idxs.at[pl.ds(0, e_per_w)])
    pltpu.sync_copy(bias_hbm, bias_v)
    bvecs = [bias_v[pl.ds(16 * j, 16)] for j in range(8)]
    lane = lax.iota(jnp.int32, 16)
    if rem:
        off = nfull * 16
        vd = idxd[pl.ds(off, 16)]
        idxd[pl.ds(off, 16)] = jnp.where(lane < rem, vd, n_nodes)
        vs = idxs[pl.ds(off, 16)]
        idxs[pl.ds(off, 16)] = jnp.where(lane < rem, vs, 0)

    def _gidx(c, tail):
        gd = idxd[pl.ds(c * 16, 16)]
        if tail:
            gd = jnp.where(lane < rem, gd, 0)
        return gd

    def stage(c, k, tail=False):
        pltpu.async_copy(a_hbm.at[_gidx(c, tail)], A[k], SG[k])
        pltpu.async_copy(b_hbm.at[idxs[pl.ds(c * 16, 16)]], B[k], SG[k])

    def wait_gather(c, k, tail=False):
        pltpu.make_async_copy(a_hbm.at[_gidx(c, tail)], A[k], SG[k]).wait()
        pltpu.make_async_copy(b_hbm.at[idxs[pl.ds(c * 16, 16)]],
                              B[k], SG[k]).wait()

    def compute(k):
        ab, bb, ob = A[k], B[k], O[k]

        def edge(i, carry):
            for j in range(8):
                sl = pl.ds(16 * j, 16)
                ob[i, sl] = jnp.maximum(ab[i, sl] + bb[i, sl] + bvecs[j],
                                        zero)
            return carry
        lax.fori_loop(0, 16, edge, 0)

    def scatter(c, k):
        pltpu.async_copy(O[k], aggr_sh.at[idxd[pl.ds(c * 16, 16)]],
                         SS[k], add=True)

    def wait_scatter(c, k):
        pltpu.make_async_copy(O[k], aggr_sh.at[idxd[pl.ds(c * 16, 16)]],
                              SS[k]).wait()

    def do_step(c, kc, do_stage, do_wait, tail=False, stage_tail=False):
        kn = (kc + 2) % 4
        wait_gather(c, kc, tail)
        compute(kc)
        scatter(c, kc)
        if do_wait:
            wait_scatter(c - 2, kn)
        if do_stage:
            stage(c + 2, kn, stage_tail)

    stage(0, 0)
    stage(1, 1)
    plsc.subcore_barrier()

    do_step(0, 0, True, False)
    do_step(1, 1, True, False)
    ngroups = (nchunks - 4) // 4
    if rem and (nchunks - 4) % 4 == 0:
        ngroups -= 1

    def group(g, carry):
        c0 = 2 + 4 * g
        for r in range(4):
            do_step(c0 + r, (2 + r) % 4, True, True)
        return carry
    lax.fori_loop(0, ngroups, group, 0)
    tail_c = nchunks - 1 if rem else -1
    for c in range(2 + 4 * ngroups, nchunks - 2):
        do_step(c, c % 4, True, True, stage_tail=(c + 2 == tail_c))
    for c in range(max(nchunks - 2, 2 + 4 * ngroups), nchunks):
        do_step(c, c % 4, False, False, tail=(c == tail_c))
    for c in range(nchunks - 4, nchunks):
        wait_scatter(c, c % 4)

    plsc.subcore_barrier()

    def wcopy(i, carry):
        r = row0 + i * 16
        pltpu.sync_copy(aggr_sh.at[pl.ds(r, 16)],
                        out_hbm.at[cid, pl.ds(r, 16)])
        return carry
    lax.fori_loop(0, n_blk, wcopy, 0)


@functools.lru_cache(maxsize=None)
def _make_edge_kernel(n_nodes, n_edges):
    e_per_w = n_edges // 32
    assert e_per_w * 32 == n_edges and e_per_w % 8 == 0
    idx_len = ((e_per_w + 15) // 16) * 16
    mesh = plsc.VectorSubcoreMesh(core_axis_name="c", subcore_axis_name="s")
    return pl.kernel(
        functools.partial(_edge_body, n_nodes, e_per_w),
        out_type=jax.ShapeDtypeStruct((2, n_nodes, H), jnp.float32),
        mesh=mesh,
        scratch_types=(
            [pltpu.VMEM_SHARED((n_nodes + 16, H), jnp.float32)]
            + [pltpu.VMEM((idx_len,), jnp.int32)] * 2
            + [pltpu.VMEM((16, H), jnp.float32)] * 8
            + [pltpu.VMEM((16, H), jnp.float32)] * 4
            + [pltpu.VMEM((H,), jnp.float32)]
            + [pltpu.SemaphoreType.DMA] * 8
        ),
    )


def _proj_ab_block(x_ref, w0_ref, b0_ref, wab_ref, oa_ref, ob_ref):
    h = jnp.maximum(
        jnp.dot(x_ref[...], w0_ref[...], preferred_element_type=jnp.float32)
        + b0_ref[...], 0.0)
    ab = jnp.dot(h, wab_ref[...], preferred_element_type=jnp.float32)
    oa_ref[...] = ab[:, :H]
    ob_ref[...] = ab[:, H:]


def _upd_ab_block(p_ref, wl_ref, bl_ref, wab_ref, oa_ref, ob_ref):
    agg = p_ref[0] + p_ref[1]
    h = jnp.maximum(
        jnp.dot(agg, wl_ref[...], preferred_element_type=jnp.float32)
        + bl_ref[...], 0.0)
    ab = jnp.dot(h, wab_ref[...], preferred_element_type=jnp.float32)
    oa_ref[...] = ab[:, :H]
    ob_ref[...] = ab[:, H:]


def _upd_mean_block(p_ref, wl_ref, bl_ref, o_ref, *, n_nodes):
    agg = p_ref[0] + p_ref[1]
    h = jnp.maximum(
        jnp.dot(agg, wl_ref[...], preferred_element_type=jnp.float32)
        + bl_ref[...], 0.0)
    i = pl.program_id(0)

    @pl.when(i == 0)
    def _():
        o_ref[...] = jnp.zeros_like(o_ref)
    o_ref[...] += jnp.sum(h, axis=0, keepdims=True) * (1.0 / n_nodes)


def _head_block(t_ref, g_ref, wmu_ref, bmu_ref, wlv_ref, blv_ref,
                mu_ref, lv_ref, f_ref):
    f = jnp.concatenate([t_ref[...], g_ref[...]], axis=1)
    f_ref[...] = f
    mu_ref[...] = jnp.dot(f, wmu_ref[...],
                          preferred_element_type=jnp.float32) + bmu_ref[...]
    lv_ref[...] = jnp.dot(f, wlv_ref[...],
                          preferred_element_type=jnp.float32) + blv_ref[...]


def _proj_ab(x, w0t, b0, wab, n):
    grid = (n // BN,)
    return pl.pallas_call(
        _proj_ab_block,
        grid=grid,
        in_specs=[
            pl.BlockSpec((BN, H), lambda i: (i, 0)),
            pl.BlockSpec((H, H), lambda i: (0, 0)),
            pl.BlockSpec((1, H), lambda i: (0, 0)),
            pl.BlockSpec((H, 2 * H), lambda i: (0, 0)),
        ],
        out_specs=[
            pl.BlockSpec((BN, H), lambda i: (i, 0)),
            pl.BlockSpec((BN, H), lambda i: (i, 0)),
        ],
        out_shape=[
            jax.ShapeDtypeStruct((n, H), jnp.float32),
            jax.ShapeDtypeStruct((n, H), jnp.float32),
        ],
    )(x, w0t, b0.reshape(1, H), wab)


def _upd_ab(p, wlt, bl, wab, n):
    grid = (n // BN,)
    return pl.pallas_call(
        _upd_ab_block,
        grid=grid,
        in_specs=[
            pl.BlockSpec((2, BN, H), lambda i: (0, i, 0)),
            pl.BlockSpec((H, H), lambda i: (0, 0)),
            pl.BlockSpec((1, H), lambda i: (0, 0)),
            pl.BlockSpec((H, 2 * H), lambda i: (0, 0)),
        ],
        out_specs=[
            pl.BlockSpec((BN, H), lambda i: (i, 0)),
            pl.BlockSpec((BN, H), lambda i: (i, 0)),
        ],
        out_shape=[
            jax.ShapeDtypeStruct((n, H), jnp.float32),
            jax.ShapeDtypeStruct((n, H), jnp.float32),
        ],
    )(p, wlt, bl.reshape(1, H), wab)


def _upd_mean(p, wlt, bl, n):
    grid = (n // BN,)
    return pl.pallas_call(
        functools.partial(_upd_mean_block, n_nodes=n),
        grid=grid,
        in_specs=[
            pl.BlockSpec((2, BN, H), lambda i: (0, i, 0)),
            pl.BlockSpec((H, H), lambda i: (0, 0)),
            pl.BlockSpec((1, H), lambda i: (0, 0)),
        ],
        out_specs=pl.BlockSpec((1, H), lambda i: (0, 0)),
        out_shape=jax.ShapeDtypeStruct((1, H), jnp.float32),
    )(p, wlt, bl.reshape(1, H))


def _heads(tvec, gvec, muW, mub, lvW, lvb):
    z = muW.shape[0]
    return pl.pallas_call(
        _head_block,
        out_shape=[
            jax.ShapeDtypeStruct((1, z), jnp.float32),
            jax.ShapeDtypeStruct((1, z), jnp.float32),
            jax.ShapeDtypeStruct((1, 2 * H), jnp.float32),
        ],
    )(tvec, gvec, muW.T, mub.reshape(1, z), lvW.T, lvb.reshape(1, z))


def _ab_weight(mW):
    return jnp.concatenate([mW[:, :H].T, mW[:, H:].T], axis=1)


def _encode_one(x, ei, inW, inb, layers, n_nodes, n_edges):
    dst = ei[1]
    src = ei[0]
    edge_k = _make_edge_kernel(n_nodes, n_edges)

    (m0W, m0b, l0W, l0b), (m1W, m1b, l1W, l1b), (m2W, m2b, l2W, l2b) = layers

    a, b = _proj_ab(x, inW.T, inb, _ab_weight(m0W), n_nodes)
    p = edge_k(a, b, dst, src, m0b)
    a, b = _upd_ab(p, l0W.T, l0b, _ab_weight(m1W), n_nodes)
    p = edge_k(a, b, dst, src, m1b)
    a, b = _upd_ab(p, l1W.T, l1b, _ab_weight(m2W), n_nodes)
    p = edge_k(a, b, dst, src, m2b)
    return _upd_mean(p, l2W.T, l2b, n_nodes)


def kernel(tree_x, graph_x, tree_edge_index, graph_edge_index,
           t_inW, t_inb, t_m0W, t_m0b, t_l0W, t_l0b, t_m1W, t_m1b,
           t_l1W, t_l1b, t_m2W, t_m2b, t_l2W, t_l2b,
           g_inW, g_inb, g_m0W, g_m0b, g_l0W, g_l0b, g_m1W, g_m1b,
           g_l1W, g_l1b, g_m2W, g_m2b, g_l2W, g_l2b,
           muW, mub, lvW, lvb):
    tlayers = [(t_m0W, t_m0b, t_l0W, t_l0b), (t_m1W, t_m1b, t_l1W, t_l1b),
               (t_m2W, t_m2b, t_l2W, t_l2b)]
    glayers = [(g_m0W, g_m0b, g_l0W, g_l0b), (g_m1W, g_m1b, g_l1W, g_l1b),
               (g_m2W, g_m2b, g_l2W, g_l2b)]
    tvec = _encode_one(tree_x, tree_edge_index, t_inW, t_inb, tlayers,
                       tree_x.shape[0], tree_edge_index.shape[1])
    gvec = _encode_one(graph_x, graph_edge_index, g_inW, g_inb, glayers,
                       graph_x.shape[0], graph_edge_index.shape[1])
    mu, logvar, fused = _heads(tvec, gvec, muW, mub, lvW, lvb)
    return (mu, logvar, fused)


# speedup vs baseline: 6.9520x; 1.5504x over previous
import functools

import jax
import jax.numpy as jnp
from jax import lax
from jax.experimental import pallas as pl
from jax.experimental.pallas import tpu as pltpu
from jax.experimental.pallas import tpu_sc as plsc

H = 128
BN = 2000


def _edge_body(n_nodes, e_per_w, a_hbm, b_hbm, dst_hbm, src_hbm, bias_hbm,
               out_hbm, aggr_sh, idxd, idxs,
               a0, a1, a2, a3, a4, a5, b0, b1, b2, b3, b4, b5, bias_v,
               sg0, sg1, sg2, sg3, sg4, sg5, ss0, ss1, ss2, ss3, ss4, ss5):
    cid = lax.axis_index("c")
    sid = lax.axis_index("s")
    zero = jnp.zeros((16,), jnp.float32)
    A = [a0, a1, a2, a3, a4, a5]
    B = [b0, b1, b2, b3, b4, b5]
    SG = [sg0, sg1, sg2, sg3, sg4, sg5]
    SS = [ss0, ss1, ss2, ss3, ss4, ss5]
    NS = 6
    D = 4
    nfull = e_per_w // 16
    rem = e_per_w - nfull * 16
    nchunks = nfull + (1 if rem else 0)

    def zrow(i, carry):
        for j in range(8):
            a0[i, pl.ds(16 * j, 16)] = zero
        return carry
    lax.fori_loop(0, 16, zrow, 0)

    rpt = ((-(-n_nodes // 16) + 15) // 16) * 16
    row0 = sid * rpt
    nrows = jnp.clip(n_nodes - row0, 0, rpt)
    n_blk = nrows // 16

    def zcopy(i, carry):
        pltpu.sync_copy(a0, aggr_sh.at[pl.ds(row0 + i * 16, 16)])
        return carry
    lax.fori_loop(0, n_blk, zcopy, 0)

    base = cid * (e_per_w * 16) + sid * e_per_w
    pltpu.sync_copy(dst_hbm.at[pl.ds(base, e_per_w)],
                    idxd.at[pl.ds(0, e_per_w)])
    pltpu.sync_copy(src_hbm.at[pl.ds(base, e_per_w)],
                    idxs.at[pl.ds(0, e_per_w)])
    pltpu.sync_copy(bias_hbm, bias_v)
    bvecs = [bias_v[pl.ds(16 * j, 16)] for j in range(8)]

    lane = lax.iota(jnp.int32, 16)
    if rem:
        off = nfull * 16
        vd = idxd[pl.ds(off, 16)]
        idxd[pl.ds(off, 16)] = jnp.where(lane < rem, vd, n_nodes)
        vs = idxs[pl.ds(off, 16)]
        idxs[pl.ds(off, 16)] = jnp.where(lane < rem, vs, 0)

    def _gidx(c, tail):
        gd = idxd[pl.ds(c * 16, 16)]
        if tail:
            gd = jnp.where(lane < rem, gd, 0)
        return gd

    def stage(c, k, tail=False):
        pltpu.async_copy(a_hbm.at[_gidx(c, tail)], A[k], SG[k])
        pltpu.async_copy(b_hbm.at[idxs[pl.ds(c * 16, 16)]], B[k], SG[k])

    def wait_gather(c, k, tail=False):
        pltpu.make_async_copy(a_hbm.at[_gidx(c, tail)], A[k], SG[k]).wait()
        pltpu.make_async_copy(b_hbm.at[idxs[pl.ds(c * 16, 16)]],
                              B[k], SG[k]).wait()

    def compute(k):
        ab, bb = A[k], B[k]

        def edge(i, carry):
            for j in range(8):
                sl = pl.ds(16 * j, 16)
                ab[i, sl] = jnp.maximum(ab[i, sl] + bb[i, sl] + bvecs[j],
                                        zero)
            return carry
        lax.fori_loop(0, 16, edge, 0)

    def scatter(c, k):
        pltpu.async_copy(A[k], aggr_sh.at[idxd[pl.ds(c * 16, 16)]],
                         SS[k], add=True)

    def wait_scatter(c, k):
        pltpu.make_async_copy(A[k], aggr_sh.at[idxd[pl.ds(c * 16, 16)]],
                              SS[k]).wait()

    def do_step(c, kc, do_stage, do_wait, tail=False, stage_tail=False):
        kn = (kc + D) % NS
        wait_gather(c, kc, tail)
        compute(kc)
        scatter(c, kc)
        if do_stage:
            if do_wait:
                wait_scatter(c - 2, kn)
            stage(c + D, kn, stage_tail)

    for c in range(D):
        stage(c, c)
    plsc.subcore_barrier()

    do_step(0, 0, True, False)
    do_step(1, 1, True, False)
    ngroups = (nchunks - 2 - D) // NS
    if rem and (nchunks - 2 - D) % NS == 0:
        ngroups -= 1

    def group(g, carry):
        c0 = 2 + NS * g
        for r in range(NS):
            do_step(c0 + r, (2 + r) % NS, True, True)
        return carry
    lax.fori_loop(0, ngroups, group, 0)
    tail_c = nchunks - 1 if rem else -1
    for c in range(2 + NS * ngroups, nchunks):
        do_step(c, c % NS, c + D < nchunks, True,
                tail=(c == tail_c), stage_tail=(c + D == tail_c))
    for c in range(nchunks - NS, nchunks):
        wait_scatter(c, c % NS)

    plsc.subcore_barrier()

    def wcopy(i, carry):
        r = row0 + i * 16
        pltpu.sync_copy(aggr_sh.at[pl.ds(r, 16)],
                        out_hbm.at[cid, pl.ds(r, 16)])
        return carry
    lax.fori_loop(0, n_blk, wcopy, 0)


@functools.lru_cache(maxsize=None)
def _make_edge_kernel(n_nodes, n_edges):
    e_per_w = n_edges // 32
    assert e_per_w * 32 == n_edges and e_per_w % 8 == 0
    idx_len = ((e_per_w + 15) // 16) * 16
    mesh = plsc.VectorSubcoreMesh(core_axis_name="c", subcore_axis_name="s")
    return pl.kernel(
        functools.partial(_edge_body, n_nodes, e_per_w),
        out_type=jax.ShapeDtypeStruct((2, n_nodes, H), jnp.float32),
        mesh=mesh,
        scratch_types=(
            [pltpu.VMEM_SHARED((n_nodes + 16, H), jnp.float32)]
            + [pltpu.VMEM((idx_len,), jnp.int32)] * 2
            + [pltpu.VMEM((16, H), jnp.float32)] * 12
            + [pltpu.VMEM((H,), jnp.float32)]
            + [pltpu.SemaphoreType.DMA] * 12
        ),
    )


def _proj_ab_block(x_ref, w0_ref, b0_ref, wab_ref, oa_ref, ob_ref):
    h = jnp.maximum(
        jnp.dot(x_ref[...], w0_ref[...], preferred_element_type=jnp.float32)
        + b0_ref[...], 0.0)
    ab = jnp.dot(h, wab_ref[...], preferred_element_type=jnp.float32)
    oa_ref[...] = ab[:, :H]
    ob_ref[...] = ab[:, H:]


def _upd_ab_block(p_ref, wl_ref, bl_ref, wab_ref, oa_ref, ob_ref):
    agg = p_ref[0] + p_ref[1]
    h = jnp.maximum(
        jnp.dot(agg, wl_ref[...], preferred_element_type=jnp.float32)
        + bl_ref[...], 0.0)
    ab = jnp.dot(h, wab_ref[...], preferred_element_type=jnp.float32)
    oa_ref[...] = ab[:, :H]
    ob_ref[...] = ab[:, H:]


def _upd_mean_block(p_ref, wl_ref, bl_ref, o_ref, *, n_nodes):
    agg = p_ref[0] + p_ref[1]
    h = jnp.maximum(
        jnp.dot(agg, wl_ref[...], preferred_element_type=jnp.float32)
        + bl_ref[...], 0.0)
    i = pl.program_id(0)

    @pl.when(i == 0)
    def _():
        o_ref[...] = jnp.zeros_like(o_ref)
    o_ref[...] += jnp.sum(h, axis=0, keepdims=True) * (1.0 / n_nodes)


def _head_block(t_ref, g_ref, wmu_ref, bmu_ref, wlv_ref, blv_ref,
                mu_ref, lv_ref, f_ref):
    f = jnp.concatenate([t_ref[...], g_ref[...]], axis=1)
    f_ref[...] = f
    mu_ref[...] = jnp.dot(f, wmu_ref[...],
                          preferred_element_type=jnp.float32) + bmu_ref[...]
    lv_ref[...] = jnp.dot(f, wlv_ref[...],
                          preferred_element_type=jnp.float32) + blv_ref[...]


def _proj_ab(x, w0t, b0, wab, n):
    grid = (n // BN,)
    return pl.pallas_call(
        _proj_ab_block,
        grid=grid,
        in_specs=[
            pl.BlockSpec((BN, H), lambda i: (i, 0)),
            pl.BlockSpec((H, H), lambda i: (0, 0)),
            pl.BlockSpec((1, H), lambda i: (0, 0)),
            pl.BlockSpec((H, 2 * H), lambda i: (0, 0)),
        ],
        out_specs=[
            pl.BlockSpec((BN, H), lambda i: (i, 0)),
            pl.BlockSpec((BN, H), lambda i: (i, 0)),
        ],
        out_shape=[
            jax.ShapeDtypeStruct((n, H), jnp.float32),
            jax.ShapeDtypeStruct((n, H), jnp.float32),
        ],
    )(x, w0t, b0.reshape(1, H), wab)


def _upd_ab(p, wlt, bl, wab, n):
    grid = (n // BN,)
    return pl.pallas_call(
        _upd_ab_block,
        grid=grid,
        in_specs=[
            pl.BlockSpec((2, BN, H), lambda i: (0, i, 0)),
            pl.BlockSpec((H, H), lambda i: (0, 0)),
            pl.BlockSpec((1, H), lambda i: (0, 0)),
            pl.BlockSpec((H, 2 * H), lambda i: (0, 0)),
        ],
        out_specs=[
            pl.BlockSpec((BN, H), lambda i: (i, 0)),
            pl.BlockSpec((BN, H), lambda i: (i, 0)),
        ],
        out_shape=[
            jax.ShapeDtypeStruct((n, H), jnp.float32),
            jax.ShapeDtypeStruct((n, H), jnp.float32),
        ],
    )(p, wlt, bl.reshape(1, H), wab)


def _upd_mean(p, wlt, bl, n):
    grid = (n // BN,)
    return pl.pallas_call(
        functools.partial(_upd_mean_block, n_nodes=n),
        grid=grid,
        in_specs=[
            pl.BlockSpec((2, BN, H), lambda i: (0, i, 0)),
            pl.BlockSpec((H, H), lambda i: (0, 0)),
            pl.BlockSpec((1, H), lambda i: (0, 0)),
        ],
        out_specs=pl.BlockSpec((1, H), lambda i: (0, 0)),
        out_shape=jax.ShapeDtypeStruct((1, H), jnp.float32),
    )(p, wlt, bl.reshape(1, H))


def _heads(tvec, gvec, muW, mub, lvW, lvb):
    z = muW.shape[0]
    return pl.pallas_call(
        _head_block,
        out_shape=[
            jax.ShapeDtypeStruct((1, z), jnp.float32),
            jax.ShapeDtypeStruct((1, z), jnp.float32),
            jax.ShapeDtypeStruct((1, 2 * H), jnp.float32),
        ],
    )(tvec, gvec, muW.T, mub.reshape(1, z), lvW.T, lvb.reshape(1, z))


def _ab_weight(mW):
    return jnp.concatenate([mW[:, :H].T, mW[:, H:].T], axis=1)


def _encode_one(x, ei, inW, inb, layers, n_nodes, n_edges):
    dst = ei[1]
    src = ei[0]
    edge_k = _make_edge_kernel(n_nodes, n_edges)

    (m0W, m0b, l0W, l0b), (m1W, m1b, l1W, l1b), (m2W, m2b, l2W, l2b) = layers

    a, b = _proj_ab(x, inW.T, inb, _ab_weight(m0W), n_nodes)
    p = edge_k(a, b, dst, src, m0b)
    a, b = _upd_ab(p, l0W.T, l0b, _ab_weight(m1W), n_nodes)
    p = edge_k(a, b, dst, src, m1b)
    a, b = _upd_ab(p, l1W.T, l1b, _ab_weight(m2W), n_nodes)
    p = edge_k(a, b, dst, src, m2b)
    return _upd_mean(p, l2W.T, l2b, n_nodes)


def kernel(tree_x, graph_x, tree_edge_index, graph_edge_index,
           t_inW, t_inb, t_m0W, t_m0b, t_l0W, t_l0b, t_m1W, t_m1b,
           t_l1W, t_l1b, t_m2W, t_m2b, t_l2W, t_l2b,
           g_inW, g_inb, g_m0W, g_m0b, g_l0W, g_l0b, g_m1W, g_m1b,
           g_l1W, g_l1b, g_m2W, g_m2b, g_l2W, g_l2b,
           muW, mub, lvW, lvb):
    tlayers = [(t_m0W, t_m0b, t_l0W, t_l0b), (t_m1W, t_m1b, t_l1W, t_l1b),
               (t_m2W, t_m2b, t_l2W, t_l2b)]
    glayers = [(g_m0W, g_m0b, g_l0W, g_l0b), (g_m1W, g_m1b, g_l1W, g_l1b),
               (g_m2W, g_m2b, g_l2W, g_l2b)]
    tvec = _encode_one(tree_x, tree_edge_index, t_inW, t_inb, tlayers,
                       tree_x.shape[0], tree_edge_index.shape[1])
    gvec = _encode_one(graph_x, graph_edge_index, g_inW, g_inb, glayers,
                       graph_x.shape[0], graph_edge_index.shape[1])
    mu, logvar, fused = _heads(tvec, gvec, muW, mub, lvW, lvb)
    return (mu, logvar, fused)


# speedup vs baseline: 7.5899x; 1.0917x over previous
import functools

import jax
import jax.numpy as jnp
from jax import lax
from jax.experimental import pallas as pl
from jax.experimental.pallas import tpu as pltpu
from jax.experimental.pallas import tpu_sc as plsc

H = 128
BN = 2000


def _edge_body(n_nodes, e_per_w, a_hbm, b_hbm, dst_hbm, src_hbm, bias_hbm,
               out_hbm, aggr_sh, idxd, idxs,
               a0, a1, a2, a3, a4, a5, a6, b0, b1, b2, b3, b4, b5, b6,
               bias_v, sg0, sg1, sg2, sg3, sg4, sg5, sg6,
               ss0, ss1, ss2, ss3, ss4, ss5, ss6):
    cid = lax.axis_index("c")
    sid = lax.axis_index("s")
    zero = jnp.zeros((16,), jnp.float32)
    A = [a0, a1, a2, a3, a4, a5, a6]
    B = [b0, b1, b2, b3, b4, b5, b6]
    SG = [sg0, sg1, sg2, sg3, sg4, sg5, sg6]
    SS = [ss0, ss1, ss2, ss3, ss4, ss5, ss6]
    NS = 7
    D = 5
    nfull = e_per_w // 16
    rem = e_per_w - nfull * 16
    nchunks = nfull + (1 if rem else 0)

    def zrow(i, carry):
        for j in range(8):
            a0[i, pl.ds(16 * j, 16)] = zero
        return carry
    lax.fori_loop(0, 16, zrow, 0)

    rpt = ((-(-n_nodes // 16) + 15) // 16) * 16
    row0 = sid * rpt
    nrows = jnp.clip(n_nodes - row0, 0, rpt)
    n_blk = nrows // 16

    def zcopy(i, carry):
        pltpu.sync_copy(a0, aggr_sh.at[pl.ds(row0 + i * 16, 16)])
        return carry
    lax.fori_loop(0, n_blk, zcopy, 0)

    base = cid * (e_per_w * 16) + sid * e_per_w
    pltpu.sync_copy(dst_hbm.at[pl.ds(base, e_per_w)],
                    idxd.at[pl.ds(0, e_per_w)])
    pltpu.sync_copy(src_hbm.at[pl.ds(base, e_per_w)],
                    idxs.at[pl.ds(0, e_per_w)])
    pltpu.sync_copy(bias_hbm, bias_v)
    bvecs = [bias_v[pl.ds(16 * j, 16)] for j in range(8)]

    lane = lax.iota(jnp.int32, 16)
    if rem:
        off = nfull * 16
        vd = idxd[pl.ds(off, 16)]
        idxd[pl.ds(off, 16)] = jnp.where(lane < rem, vd, n_nodes)
        vs = idxs[pl.ds(off, 16)]
        idxs[pl.ds(off, 16)] = jnp.where(lane < rem, vs, 0)

    def _gidx(c, tail):
        gd = idxd[pl.ds(c * 16, 16)]
        if tail:
            gd = jnp.where(lane < rem, gd, 0)
        return gd

    def stage(c, k, tail=False):
        pltpu.async_copy(a_hbm.at[_gidx(c, tail)], A[k], SG[k])
        pltpu.async_copy(b_hbm.at[idxs[pl.ds(c * 16, 16)]], B[k], SG[k])

    def wait_gather(c, k, tail=False):
        pltpu.make_async_copy(a_hbm.at[_gidx(c, tail)], A[k], SG[k]).wait()
        pltpu.make_async_copy(b_hbm.at[idxs[pl.ds(c * 16, 16)]],
                              B[k], SG[k]).wait()

    def compute(k):
        ab, bb = A[k], B[k]

        def edge(i, carry):
            for j in range(8):
                sl = pl.ds(16 * j, 16)
                ab[i, sl] = jnp.maximum(ab[i, sl] + bb[i, sl] + bvecs[j],
                                        zero)
            return carry
        lax.fori_loop(0, 16, edge, 0)

    def scatter(c, k):
        pltpu.async_copy(A[k], aggr_sh.at[idxd[pl.ds(c * 16, 16)]],
                         SS[k], add=True)

    def wait_scatter(c, k):
        pltpu.make_async_copy(A[k], aggr_sh.at[idxd[pl.ds(c * 16, 16)]],
                              SS[k]).wait()

    def do_step(c, kc, do_stage, do_wait, tail=False, stage_tail=False):
        kn = (kc + D) % NS
        wait_gather(c, kc, tail)
        compute(kc)
        scatter(c, kc)
        if do_stage:
            if do_wait:
                wait_scatter(c - 2, kn)
            stage(c + D, kn, stage_tail)

    for c in range(D):
        stage(c, c)
    plsc.subcore_barrier()

    do_step(0, 0, True, False)
    do_step(1, 1, True, False)
    ngroups = (nchunks - 2 - D) // NS
    if rem and (nchunks - 2 - D) % NS == 0:
        ngroups -= 1

    def group(g, carry):
        c0 = 2 + NS * g
        for r in range(NS):
            do_step(c0 + r, (2 + r) % NS, True, True)
        return carry
    lax.fori_loop(0, ngroups, group, 0)
    tail_c = nchunks - 1 if rem else -1
    for c in range(2 + NS * ngroups, nchunks):
        do_step(c, c % NS, c + D < nchunks, True,
                tail=(c == tail_c), stage_tail=(c + D == tail_c))
    for c in range(nchunks - NS, nchunks):
        wait_scatter(c, c % NS)

    plsc.subcore_barrier()

    def wcopy(i, carry):
        r = row0 + i * 16
        pltpu.sync_copy(aggr_sh.at[pl.ds(r, 16)],
                        out_hbm.at[cid, pl.ds(r, 16)])
        return carry
    lax.fori_loop(0, n_blk, wcopy, 0)


@functools.lru_cache(maxsize=None)
def _make_edge_kernel(n_nodes, n_edges):
    e_per_w = n_edges // 32
    assert e_per_w * 32 == n_edges and e_per_w % 8 == 0
    idx_len = ((e_per_w + 15) // 16) * 16
    mesh = plsc.VectorSubcoreMesh(core_axis_name="c", subcore_axis_name="s")
    return pl.kernel(
        functools.partial(_edge_body, n_nodes, e_per_w),
        out_type=jax.ShapeDtypeStruct((2, n_nodes, H), jnp.float32),
        mesh=mesh,
        scratch_types=(
            [pltpu.VMEM_SHARED((n_nodes + 16, H), jnp.float32)]
            + [pltpu.VMEM((idx_len,), jnp.int32)] * 2
            + [pltpu.VMEM((16, H), jnp.float32)] * 14
            + [pltpu.VMEM((H,), jnp.float32)]
            + [pltpu.SemaphoreType.DMA] * 14
        ),
    )


def _proj_ab_block(x_ref, w0_ref, b0_ref, wab_ref, oa_ref, ob_ref):
    h = jnp.maximum(
        jnp.dot(x_ref[...], w0_ref[...], preferred_element_type=jnp.float32)
        + b0_ref[...], 0.0)
    ab = jnp.dot(h, wab_ref[...], preferred_element_type=jnp.float32)
    oa_ref[...] = ab[:, :H]
    ob_ref[...] = ab[:, H:]


def _upd_ab_block(p_ref, wl_ref, bl_ref, wab_ref, oa_ref, ob_ref):
    agg = p_ref[0] + p_ref[1]
    h = jnp.maximum(
        jnp.dot(agg, wl_ref[...], preferred_element_type=jnp.float32)
        + bl_ref[...], 0.0)
    ab = jnp.dot(h, wab_ref[...], preferred_element_type=jnp.float32)
    oa_ref[...] = ab[:, :H]
    ob_ref[...] = ab[:, H:]


def _upd_mean_block(p_ref, wl_ref, bl_ref, o_ref, *, n_nodes):
    agg = p_ref[0] + p_ref[1]
    h = jnp.maximum(
        jnp.dot(agg, wl_ref[...], preferred_element_type=jnp.float32)
        + bl_ref[...], 0.0)
    i = pl.program_id(0)

    @pl.when(i == 0)
    def _():
        o_ref[...] = jnp.zeros_like(o_ref)
    o_ref[...] += jnp.sum(h, axis=0, keepdims=True) * (1.0 / n_nodes)


def _head_block(t_ref, g_ref, wmu_ref, bmu_ref, wlv_ref, blv_ref,
                mu_ref, lv_ref, f_ref):
    f = jnp.concatenate([t_ref[...], g_ref[...]], axis=1)
    f_ref[...] = f
    mu_ref[...] = jnp.dot(f, wmu_ref[...],
                          preferred_element_type=jnp.float32) + bmu_ref[...]
    lv_ref[...] = jnp.dot(f, wlv_ref[...],
                          preferred_element_type=jnp.float32) + blv_ref[...]


def _proj_ab(x, w0t, b0, wab, n):
    grid = (n // BN,)
    return pl.pallas_call(
        _proj_ab_block,
        grid=grid,
        in_specs=[
            pl.BlockSpec((BN, H), lambda i: (i, 0)),
            pl.BlockSpec((H, H), lambda i: (0, 0)),
            pl.BlockSpec((1, H), lambda i: (0, 0)),
            pl.BlockSpec((H, 2 * H), lambda i: (0, 0)),
        ],
        out_specs=[
            pl.BlockSpec((BN, H), lambda i: (i, 0)),
            pl.BlockSpec((BN, H), lambda i: (i, 0)),
        ],
        out_shape=[
            jax.ShapeDtypeStruct((n, H), jnp.float32),
            jax.ShapeDtypeStruct((n, H), jnp.float32),
        ],
    )(x, w0t, b0.reshape(1, H), wab)


def _upd_ab(p, wlt, bl, wab, n):
    grid = (n // BN,)
    return pl.pallas_call(
        _upd_ab_block,
        grid=grid,
        in_specs=[
            pl.BlockSpec((2, BN, H), lambda i: (0, i, 0)),
            pl.BlockSpec((H, H), lambda i: (0, 0)),
            pl.BlockSpec((1, H), lambda i: (0, 0)),
            pl.BlockSpec((H, 2 * H), lambda i: (0, 0)),
        ],
        out_specs=[
            pl.BlockSpec((BN, H), lambda i: (i, 0)),
            pl.BlockSpec((BN, H), lambda i: (i, 0)),
        ],
        out_shape=[
            jax.ShapeDtypeStruct((n, H), jnp.float32),
            jax.ShapeDtypeStruct((n, H), jnp.float32),
        ],
    )(p, wlt, bl.reshape(1, H), wab)


def _upd_mean(p, wlt, bl, n):
    grid = (n // BN,)
    return pl.pallas_call(
        functools.partial(_upd_mean_block, n_nodes=n),
        grid=grid,
        in_specs=[
            pl.BlockSpec((2, BN, H), lambda i: (0, i, 0)),
            pl.BlockSpec((H, H), lambda i: (0, 0)),
            pl.BlockSpec((1, H), lambda i: (0, 0)),
        ],
        out_specs=pl.BlockSpec((1, H), lambda i: (0, 0)),
        out_shape=jax.ShapeDtypeStruct((1, H), jnp.float32),
    )(p, wlt, bl.reshape(1, H))


def _heads(tvec, gvec, muW, mub, lvW, lvb):
    z = muW.shape[0]
    return pl.pallas_call(
        _head_block,
        out_shape=[
            jax.ShapeDtypeStruct((1, z), jnp.float32),
            jax.ShapeDtypeStruct((1, z), jnp.float32),
            jax.ShapeDtypeStruct((1, 2 * H), jnp.float32),
        ],
    )(tvec, gvec, muW.T, mub.reshape(1, z), lvW.T, lvb.reshape(1, z))


def _ab_weight(mW):
    return jnp.concatenate([mW[:, :H].T, mW[:, H:].T], axis=1)


def _encode_one(x, ei, inW, inb, layers, n_nodes, n_edges):
    dst = ei[1]
    src = ei[0]
    edge_k = _make_edge_kernel(n_nodes, n_edges)

    (m0W, m0b, l0W, l0b), (m1W, m1b, l1W, l1b), (m2W, m2b, l2W, l2b) = layers

    a, b = _proj_ab(x, inW.T, inb, _ab_weight(m0W), n_nodes)
    p = edge_k(a, b, dst, src, m0b)
    a, b = _upd_ab(p, l0W.T, l0b, _ab_weight(m1W), n_nodes)
    p = edge_k(a, b, dst, src, m1b)
    a, b = _upd_ab(p, l1W.T, l1b, _ab_weight(m2W), n_nodes)
    p = edge_k(a, b, dst, src, m2b)
    return _upd_mean(p, l2W.T, l2b, n_nodes)


def kernel(tree_x, graph_x, tree_edge_index, graph_edge_index,
           t_inW, t_inb, t_m0W, t_m0b, t_l0W, t_l0b, t_m1W, t_m1b,
           t_l1W, t_l1b, t_m2W, t_m2b, t_l2W, t_l2b,
           g_inW, g_inb, g_m0W, g_m0b, g_l0W, g_l0b, g_m1W, g_m1b,
           g_l1W, g_l1b, g_m2W, g_m2b, g_l2W, g_l2b,
           muW, mub, lvW, lvb):
    tlayers = [(t_m0W, t_m0b, t_l0W, t_l0b), (t_m1W, t_m1b, t_l1W, t_l1b),
               (t_m2W, t_m2b, t_l2W, t_l2b)]
    glayers = [(g_m0W, g_m0b, g_l0W, g_l0b), (g_m1W, g_m1b, g_l1W, g_l1b),
               (g_m2W, g_m2b, g_l2W, g_l2b)]
    tvec = _encode_one(tree_x, tree_edge_index, t_inW, t_inb, tlayers,
                       tree_x.shape[0], tree_edge_index.shape[1])
    gvec = _encode_one(graph_x, graph_edge_index, g_inW, g_inb, glayers,
                       graph_x.shape[0], graph_edge_index.shape[1])
    mu, logvar, fused = _heads(tvec, gvec, muW, mub, lvW, lvb)
    return (mu, logvar, fused)


# speedup vs baseline: 8.3056x; 1.0943x over previous
import functools

import jax
import jax.numpy as jnp
from jax import lax
from jax.experimental import pallas as pl
from jax.experimental.pallas import tpu as pltpu
from jax.experimental.pallas import tpu_sc as plsc

H = 128
BN = 2000


def _edge_body(n_nodes, e_per_w, a_hbm, b_hbm, dst_hbm, src_hbm, bias_hbm,
               zeros_hbm, out_hbm, aggr_sh, idxd, idxs,
               a0, a1, a2, a3, a4, a5, a6, b0, b1, b2, b3, b4, b5, b6,
               bias_v, sg0, sg1, sg2, sg3, sg4, sg5, sg6,
               ss0, ss1, ss2, ss3, ss4, ss5, ss6):
    cid = lax.axis_index("c")
    sid = lax.axis_index("s")
    zero = jnp.zeros((16,), jnp.float32)
    A = [a0, a1, a2, a3, a4, a5, a6]
    B = [b0, b1, b2, b3, b4, b5, b6]
    SG = [sg0, sg1, sg2, sg3, sg4, sg5, sg6]
    SS = [ss0, ss1, ss2, ss3, ss4, ss5, ss6]
    NS = 7
    D = 5
    nfull = e_per_w // 16
    rem = e_per_w - nfull * 16
    nchunks = nfull + (1 if rem else 0)

    rpt = ((-(-n_nodes // 16) + 15) // 16) * 16
    row0 = sid * rpt
    full_tiles = n_nodes // rpt
    last_rows = n_nodes - full_tiles * rpt

    @pl.when(sid < full_tiles)
    def _():
        pltpu.sync_copy(zeros_hbm, aggr_sh.at[pl.ds(row0, rpt)])
    if last_rows:
        @pl.when(sid == full_tiles)
        def _():
            pltpu.sync_copy(zeros_hbm.at[pl.ds(0, last_rows)],
                            aggr_sh.at[pl.ds(row0, last_rows)])

    base = cid * (e_per_w * 16) + sid * e_per_w
    pltpu.sync_copy(dst_hbm.at[pl.ds(base, e_per_w)],
                    idxd.at[pl.ds(0, e_per_w)])
    pltpu.sync_copy(src_hbm.at[pl.ds(base, e_per_w)],
                    idxs.at[pl.ds(0, e_per_w)])
    pltpu.sync_copy(bias_hbm, bias_v)
    bvecs = [bias_v[pl.ds(16 * j, 16)] for j in range(8)]

    lane = lax.iota(jnp.int32, 16)
    if rem:
        off = nfull * 16
        vd = idxd[pl.ds(off, 16)]
        idxd[pl.ds(off, 16)] = jnp.where(lane < rem, vd, n_nodes)
        vs = idxs[pl.ds(off, 16)]
        idxs[pl.ds(off, 16)] = jnp.where(lane < rem, vs, 0)

    def _gidx(c, tail):
        gd = idxd[pl.ds(c * 16, 16)]
        if tail:
            gd = jnp.where(lane < rem, gd, 0)
        return gd

    def stage(c, k, tail=False):
        pltpu.async_copy(a_hbm.at[_gidx(c, tail)], A[k], SG[k])
        pltpu.async_copy(b_hbm.at[idxs[pl.ds(c * 16, 16)]], B[k], SG[k])

    def wait_gather(c, k, tail=False):
        pltpu.make_async_copy(a_hbm.at[_gidx(c, tail)], A[k], SG[k]).wait()
        pltpu.make_async_copy(b_hbm.at[idxs[pl.ds(c * 16, 16)]],
                              B[k], SG[k]).wait()

    def compute(k):
        ab, bb = A[k], B[k]

        def edge(i, carry):
            for j in range(8):
                sl = pl.ds(16 * j, 16)
                ab[i, sl] = jnp.maximum(ab[i, sl] + bb[i, sl] + bvecs[j],
                                        zero)
            return carry
        lax.fori_loop(0, 16, edge, 0)

    def scatter(c, k):
        pltpu.async_copy(A[k], aggr_sh.at[idxd[pl.ds(c * 16, 16)]],
                         SS[k], add=True)

    def wait_scatter(c, k):
        pltpu.make_async_copy(A[k], aggr_sh.at[idxd[pl.ds(c * 16, 16)]],
                              SS[k]).wait()

    def do_step(c, kc, do_stage, do_wait, tail=False, stage_tail=False):
        kn = (kc + D) % NS
        wait_gather(c, kc, tail)
        compute(kc)
        scatter(c, kc)
        if do_stage:
            if do_wait:
                wait_scatter(c - 2, kn)
            stage(c + D, kn, stage_tail)

    for c in range(D):
        stage(c, c)
    plsc.subcore_barrier()

    do_step(0, 0, True, False)
    do_step(1, 1, True, False)
    ngroups = (nchunks - 2 - D) // NS
    if rem and (nchunks - 2 - D) % NS == 0:
        ngroups -= 1

    def group(g, carry):
        c0 = 2 + NS * g
        for r in range(NS):
            do_step(c0 + r, (2 + r) % NS, True, True)
        return carry
    lax.fori_loop(0, ngroups, group, 0)
    tail_c = nchunks - 1 if rem else -1
    for c in range(2 + NS * ngroups, nchunks):
        do_step(c, c % NS, c + D < nchunks, True,
                tail=(c == tail_c), stage_tail=(c + D == tail_c))
    for c in range(nchunks - NS, nchunks):
        wait_scatter(c, c % NS)

    plsc.subcore_barrier()

    @pl.when(sid < full_tiles)
    def _():
        pltpu.sync_copy(aggr_sh.at[pl.ds(row0, rpt)],
                        out_hbm.at[cid, pl.ds(row0, rpt)])
    if last_rows:
        @pl.when(sid == full_tiles)
        def _():
            pltpu.sync_copy(aggr_sh.at[pl.ds(row0, last_rows)],
                            out_hbm.at[cid, pl.ds(row0, last_rows)])


@functools.lru_cache(maxsize=None)
def _make_edge_kernel(n_nodes, n_edges):
    e_per_w = n_edges // 32
    assert e_per_w * 32 == n_edges and e_per_w % 8 == 0
    idx_len = ((e_per_w + 15) // 16) * 16
    mesh = plsc.VectorSubcoreMesh(core_axis_name="c", subcore_axis_name="s")
    return pl.kernel(
        functools.partial(_edge_body, n_nodes, e_per_w),
        out_type=jax.ShapeDtypeStruct((2, n_nodes, H), jnp.float32),
        mesh=mesh,
        scratch_types=(
            [pltpu.VMEM_SHARED((n_nodes + 16, H), jnp.float32)]
            + [pltpu.VMEM((idx_len,), jnp.int32)] * 2
            + [pltpu.VMEM((16, H), jnp.float32)] * 14
            + [pltpu.VMEM((H,), jnp.float32)]
            + [pltpu.SemaphoreType.DMA] * 14
        ),
    )


def _proj_ab_block(x_ref, w0_ref, b0_ref, wab_ref, oa_ref, ob_ref):
    h = jnp.maximum(
        jnp.dot(x_ref[...], w0_ref[...], preferred_element_type=jnp.float32)
        + b0_ref[...], 0.0)
    ab = jnp.dot(h, wab_ref[...], preferred_element_type=jnp.float32)
    oa_ref[...] = ab[:, :H]
    ob_ref[...] = ab[:, H:]


def _upd_ab_block(p_ref, wl_ref, bl_ref, wab_ref, oa_ref, ob_ref):
    agg = p_ref[0] + p_ref[1]
    h = jnp.maximum(
        jnp.dot(agg, wl_ref[...], preferred_element_type=jnp.float32)
        + bl_ref[...], 0.0)
    ab = jnp.dot(h, wab_ref[...], preferred_element_type=jnp.float32)
    oa_ref[...] = ab[:, :H]
    ob_ref[...] = ab[:, H:]


def _upd_mean_block(p_ref, wl_ref, bl_ref, o_ref, *, n_nodes):
    agg = p_ref[0] + p_ref[1]
    h = jnp.maximum(
        jnp.dot(agg, wl_ref[...], preferred_element_type=jnp.float32)
        + bl_ref[...], 0.0)
    i = pl.program_id(0)

    @pl.when(i == 0)
    def _():
        o_ref[...] = jnp.zeros_like(o_ref)
    o_ref[...] += jnp.sum(h, axis=0, keepdims=True) * (1.0 / n_nodes)


def _head_block(t_ref, g_ref, wmu_ref, bmu_ref, wlv_ref, blv_ref,
                mu_ref, lv_ref, f_ref):
    f = jnp.concatenate([t_ref[...], g_ref[...]], axis=1)
    f_ref[...] = f
    mu_ref[...] = jnp.dot(f, wmu_ref[...],
                          preferred_element_type=jnp.float32) + bmu_ref[...]
    lv_ref[...] = jnp.dot(f, wlv_ref[...],
                          preferred_element_type=jnp.float32) + blv_ref[...]


def _proj_ab(x, w0t, b0, wab, n):
    grid = (n // BN,)
    return pl.pallas_call(
        _proj_ab_block,
        grid=grid,
        in_specs=[
            pl.BlockSpec((BN, H), lambda i: (i, 0)),
            pl.BlockSpec((H, H), lambda i: (0, 0)),
            pl.BlockSpec((1, H), lambda i: (0, 0)),
            pl.BlockSpec((H, 2 * H), lambda i: (0, 0)),
        ],
        out_specs=[
            pl.BlockSpec((BN, H), lambda i: (i, 0)),
            pl.BlockSpec((BN, H), lambda i: (i, 0)),
        ],
        out_shape=[
            jax.ShapeDtypeStruct((n, H), jnp.float32),
            jax.ShapeDtypeStruct((n, H), jnp.float32),
        ],
    )(x, w0t, b0.reshape(1, H), wab)


def _upd_ab(p, wlt, bl, wab, n):
    grid = (n // BN,)
    return pl.pallas_call(
        _upd_ab_block,
        grid=grid,
        in_specs=[
            pl.BlockSpec((2, BN, H), lambda i: (0, i, 0)),
            pl.BlockSpec((H, H), lambda i: (0, 0)),
            pl.BlockSpec((1, H), lambda i: (0, 0)),
            pl.BlockSpec((H, 2 * H), lambda i: (0, 0)),
        ],
        out_specs=[
            pl.BlockSpec((BN, H), lambda i: (i, 0)),
            pl.BlockSpec((BN, H), lambda i: (i, 0)),
        ],
        out_shape=[
            jax.ShapeDtypeStruct((n, H), jnp.float32),
            jax.ShapeDtypeStruct((n, H), jnp.float32),
        ],
    )(p, wlt, bl.reshape(1, H), wab)


def _upd_mean(p, wlt, bl, n):
    grid = (n // BN,)
    return pl.pallas_call(
        functools.partial(_upd_mean_block, n_nodes=n),
        grid=grid,
        in_specs=[
            pl.BlockSpec((2, BN, H), lambda i: (0, i, 0)),
            pl.BlockSpec((H, H), lambda i: (0, 0)),
            pl.BlockSpec((1, H), lambda i: (0, 0)),
        ],
        out_specs=pl.BlockSpec((1, H), lambda i: (0, 0)),
        out_shape=jax.ShapeDtypeStruct((1, H), jnp.float32),
    )(p, wlt, bl.reshape(1, H))


def _heads(tvec, gvec, muW, mub, lvW, lvb):
    z = muW.shape[0]
    return pl.pallas_call(
        _head_block,
        out_shape=[
            jax.ShapeDtypeStruct((1, z), jnp.float32),
            jax.ShapeDtypeStruct((1, z), jnp.float32),
            jax.ShapeDtypeStruct((1, 2 * H), jnp.float32),
        ],
    )(tvec, gvec, muW.T, mub.reshape(1, z), lvW.T, lvb.reshape(1, z))


def _ab_weight(mW):
    return jnp.concatenate([mW[:, :H].T, mW[:, H:].T], axis=1)


def _encode_one(x, ei, inW, inb, layers, n_nodes, n_edges):
    dst = ei[1]
    src = ei[0]
    edge_k = _make_edge_kernel(n_nodes, n_edges)
    rpt = ((-(-n_nodes // 16) + 15) // 16) * 16
    zrows = jnp.zeros((rpt, H), jnp.float32)

    (m0W, m0b, l0W, l0b), (m1W, m1b, l1W, l1b), (m2W, m2b, l2W, l2b) = layers

    a, b = _proj_ab(x, inW.T, inb, _ab_weight(m0W), n_nodes)
    p = edge_k(a, b, dst, src, m0b, zrows)
    a, b = _upd_ab(p, l0W.T, l0b, _ab_weight(m1W), n_nodes)
    p = edge_k(a, b, dst, src, m1b, zrows)
    a, b = _upd_ab(p, l1W.T, l1b, _ab_weight(m2W), n_nodes)
    p = edge_k(a, b, dst, src, m2b, zrows)
    return _upd_mean(p, l2W.T, l2b, n_nodes)


def kernel(tree_x, graph_x, tree_edge_index, graph_edge_index,
           t_inW, t_inb, t_m0W, t_m0b, t_l0W, t_l0b, t_m1W, t_m1b,
           t_l1W, t_l1b, t_m2W, t_m2b, t_l2W, t_l2b,
           g_inW, g_inb, g_m0W, g_m0b, g_l0W, g_l0b, g_m1W, g_m1b,
           g_l1W, g_l1b, g_m2W, g_m2b, g_l2W, g_l2b,
           muW, mub, lvW, lvb):
    tlayers = [(t_m0W, t_m0b, t_l0W, t_l0b), (t_m1W, t_m1b, t_l1W, t_l1b),
               (t_m2W, t_m2b, t_l2W, t_l2b)]
    glayers = [(g_m0W, g_m0b, g_l0W, g_l0b), (g_m1W, g_m1b, g_l1W, g_l1b),
               (g_m2W, g_m2b, g_l2W, g_l2b)]
    tvec = _encode_one(tree_x, tree_edge_index, t_inW, t_inb, tlayers,
                       tree_x.shape[0], tree_edge_index.shape[1])
    gvec = _encode_one(graph_x, graph_edge_index, g_inW, g_inb, glayers,
                       graph_x.shape[0], graph_edge_index.shape[1])
    mu, logvar, fused = _heads(tvec, gvec, muW, mub, lvW, lvb)
    return (mu, logvar, fused)
